# baseline (speedup 1.0000x reference)
import functools

import numpy as np

import jax
import jax.numpy as jnp
from jax.experimental import pallas as pl
from jax.experimental.pallas import tpu as pltpu

_NUM_CLASSES = 9
_IN_CH = 103
_P = 5
_PP = _P * _P
_KF = _IN_CH * _PP
_NG = 17
_BN_EPS = 1e-5


def _round_up(v, m):
    return (v + m - 1) // m * m


def _conv1a_indicator():
    ind = np.zeros((9, _PP, 9), np.float32)
    for oi in range(3):
        for oj in range(3):
            o = oi * 3 + oj
            for ki in range(3):
                for kj in range(3):
                    t = ki * 3 + kj
                    p = (oi + ki) * _P + (oj + kj)
                    ind[t, p, o] = 1.0
    return ind


def _fold_weights(w1a, w1b):
    w1a_r = w1a.transpose(2, 3, 1, 0).reshape(9, _IN_CH, 64)
    ind = jnp.asarray(_conv1a_indicator())
    a = jnp.einsum("tcj,tpo->cpoj", w1a_r, ind)
    w1b_r = w1b.reshape(64, _IN_CH).T
    b = jnp.einsum("cj,pq->cpqj", w1b_r, jnp.eye(_PP, dtype=w1b.dtype))

    groups = []
    for m in range(9):
        groups.append(a[:, :, m, :])
        groups.append(b[:, :, m, :])
    for q in range(9, _PP):
        groups.append(b[:, :, q, :])
    w3 = jnp.concatenate(groups, axis=-1)
    return w3.reshape(_KF, _NG * 128).astype(jnp.bfloat16)


def _mega_kernel(x_ref, w_ref, bias_ref, feat_ref, ssum_ref, ssq_ref):
    tn = feat_ref.shape[0]
    xb = x_ref[...].astype(jnp.bfloat16)
    y = jnp.dot(xb, w_ref[...], preferred_element_type=jnp.float32)

    a = y[:, 0:128]
    for m in range(1, 9):
        a = jnp.maximum(a, y[:, m * 128:(m + 1) * 128])
    bmax = y[:, 9 * 128:10 * 128]
    for m in range(10, _NG):
        bmax = jnp.maximum(bmax, y[:, m * 128:(m + 1) * 128])
    c = jnp.maximum(bmax, pltpu.roll(bmax, 64, 1))
    lane = jax.lax.broadcasted_iota(jnp.int32, (tn, 128), 1)
    feat = jnp.where(lane < 64, a, jnp.maximum(a, c)) + bias_ref[...]

    feat_ref[...] = feat.astype(feat_ref.dtype)
    f3 = feat.reshape(tn // 8, 8, 128)
    ssum_ref[...] = jnp.sum(f3, axis=0)
    ssq_ref[...] = jnp.sum(f3 * f3, axis=0)


def _head_kernel(n_real, n_fake, feat_ref, ssum_ref, ssq_ref, bias_ref,
                 gamma_ref, beta_ref, w2a_ref, b2a_ref, w2b_ref, b2b_ref,
                 wfc_ref, bfc_ref, out_ref):
    bias = bias_ref[...]
    s1 = jnp.sum(ssum_ref[...], axis=0, keepdims=True) - n_fake * bias
    s2 = jnp.sum(ssq_ref[...], axis=0, keepdims=True) - n_fake * bias * bias
    mean = s1 / n_real
    var = jnp.maximum(s2 / n_real - mean * mean, 0.0)
    inv = jax.lax.rsqrt(var + _BN_EPS)
    scale = gamma_ref[...] * inv
    shift = beta_ref[...] - mean * scale

    feat = feat_ref[...].astype(jnp.float32)
    h = jnp.maximum(feat * scale + shift, 0.0)
    h = jnp.dot(h.astype(jnp.bfloat16), w2a_ref[...],
                preferred_element_type=jnp.float32) + b2a_ref[...]
    h = jnp.maximum(h, 0.0)
    h = jnp.dot(h.astype(jnp.bfloat16), w2b_ref[...],
                preferred_element_type=jnp.float32) + b2b_ref[...]
    res = feat + h
    out = jnp.dot(res.astype(jnp.bfloat16), wfc_ref[...],
                  preferred_element_type=jnp.float32) + bfc_ref[...]
    out_ref[...] = out[:, :_NUM_CLASSES].astype(out_ref.dtype)


@functools.partial(jax.jit, static_argnames=("tile_n",))
def _forward(x, w1a, b1a, w1b, b1b, gamma, beta, w2a, b2a, w2b, b2b,
             wfc, bfc, tile_n=512):
    n = x.shape[0]
    tn = min(tile_n, _round_up(n, 8))
    if n > 8 and _round_up(n, tn) // tn < 2:
        tn = _round_up((n + 1) // 2, 8)
    n_pad = _round_up(n, tn)
    grid_n = n_pad // tn

    x_flat = x.reshape(n, _KF)
    if n_pad != n:
        x_flat = jnp.pad(x_flat, ((0, n_pad - n), (0, 0)))

    w_big = _fold_weights(w1a, w1b)
    bias = jnp.concatenate([b1a, b1b]).reshape(1, 128).astype(jnp.float32)

    cparams = pltpu.CompilerParams(
        dimension_semantics=("parallel",),
        vmem_limit_bytes=100 << 20,
    )

    feat, ssum, ssq = pl.pallas_call(
        _mega_kernel,
        out_shape=(
            jax.ShapeDtypeStruct((n_pad, 128), jnp.bfloat16),
            jax.ShapeDtypeStruct((grid_n * 8, 128), jnp.float32),
            jax.ShapeDtypeStruct((grid_n * 8, 128), jnp.float32),
        ),
        grid=(grid_n,),
        in_specs=[
            pl.BlockSpec((tn, _KF), lambda i: (i, 0)),
            pl.BlockSpec((_KF, _NG * 128), lambda i: (0, 0)),
            pl.BlockSpec((1, 128), lambda i: (0, 0)),
        ],
        out_specs=(
            pl.BlockSpec((tn, 128), lambda i: (i, 0)),
            pl.BlockSpec((8, 128), lambda i: (i, 0)),
            pl.BlockSpec((8, 128), lambda i: (i, 0)),
        ),
        compiler_params=cparams,
    )(x_flat, w_big, bias)

    w2a_p = w2a.reshape(128, 128).T.astype(jnp.bfloat16)
    w2b_p = w2b.reshape(128, 128).T.astype(jnp.bfloat16)
    b2a_p = b2a.reshape(1, 128).astype(jnp.float32)
    b2b_p = b2b.reshape(1, 128).astype(jnp.float32)
    wfc_p = jnp.pad(wfc.T, ((0, 0), (0, 128 - _NUM_CLASSES))).astype(jnp.bfloat16)
    bfc_p = jnp.pad(bfc.reshape(1, -1), ((0, 0), (0, 128 - _NUM_CLASSES))
                    ).astype(jnp.float32)
    gamma_p = gamma.reshape(1, 128).astype(jnp.float32)
    beta_p = beta.reshape(1, 128).astype(jnp.float32)

    kern = functools.partial(_head_kernel, float(n), float(n_pad - n))
    out = pl.pallas_call(
        kern,
        out_shape=jax.ShapeDtypeStruct((n_pad, _NUM_CLASSES), jnp.bfloat16),
        grid=(grid_n,),
        in_specs=[
            pl.BlockSpec((tn, 128), lambda i: (i, 0)),
            pl.BlockSpec((grid_n * 8, 128), lambda i: (0, 0)),
            pl.BlockSpec((grid_n * 8, 128), lambda i: (0, 0)),
            pl.BlockSpec((1, 128), lambda i: (0, 0)),
            pl.BlockSpec((1, 128), lambda i: (0, 0)),
            pl.BlockSpec((1, 128), lambda i: (0, 0)),
            pl.BlockSpec((128, 128), lambda i: (0, 0)),
            pl.BlockSpec((1, 128), lambda i: (0, 0)),
            pl.BlockSpec((128, 128), lambda i: (0, 0)),
            pl.BlockSpec((1, 128), lambda i: (0, 0)),
            pl.BlockSpec((128, 128), lambda i: (0, 0)),
            pl.BlockSpec((1, 128), lambda i: (0, 0)),
        ],
        out_specs=pl.BlockSpec((tn, _NUM_CLASSES), lambda i: (i, 0)),
        compiler_params=cparams,
    )(feat, ssum, ssq, bias, gamma_p, beta_p, w2a_p, b2a_p, w2b_p, b2b_p,
      wfc_p, bfc_p)

    return out[:n].astype(jnp.float32)


def kernel(x, w1a, b1a, w1b, b1b, gamma, beta, w2a, b2a, w2b, b2b, wfc, bfc):
    return _forward(x, w1a, b1a, w1b, b1b, gamma, beta, w2a, b2a, w2b, b2b,
                    wfc, bfc, tile_n=512)


# speedup vs baseline: 5.4484x; 5.4484x over previous
import functools

import numpy as np

import jax
import jax.numpy as jnp
from jax.experimental import pallas as pl
from jax.experimental.pallas import tpu as pltpu

_NUM_CLASSES = 9
_IN_CH = 103
_P = 5
_PP = _P * _P
_BN_EPS = 1e-5


def _round_up(v, m):
    return (v + m - 1) // m * m


def _conv1a_indicator():
    ind = np.zeros((9, _PP, 9), np.float32)
    for oi in range(3):
        for oj in range(3):
            o = oi * 3 + oj
            for ki in range(3):
                for kj in range(3):
                    ind[ki * 3 + kj, (oi + ki) * _P + (oj + kj), o] = 1.0
    return ind


_CPAD = 112


def _feat_kernel(x_ref, w1a_ref, w1b_ref, bias_ref,
                 feat_ref, s1_ref, s2_ref, xs_ref):
    tn = feat_ref.shape[1]

    zpad = jnp.zeros((_CPAD - _IN_CH, tn), jnp.bfloat16)
    xb = []
    for p in range(_PP):
        xp = x_ref[p].astype(jnp.bfloat16)
        xb.append(xp)
        xs_ref[p * _CPAD:p * _CPAD + _IN_CH, :] = xp
        xs_ref[p * _CPAD + _IN_CH:(p + 1) * _CPAD, :] = zpad
    xs = xs_ref[...]

    y1a = jax.lax.dot_general(
        w1a_ref[...], xs, (((0,), (0,)), ((), ())),
        preferred_element_type=jnp.float32)
    m1a = y1a[0:64]
    for o in range(1, 9):
        m1a = jnp.maximum(m1a, y1a[o * 64:(o + 1) * 64])

    w1b = w1b_ref[...]
    m1b = None
    for p in range(_PP):
        z = jax.lax.dot_general(
            w1b, xb[p], (((0,), (0,)), ((), ())),
            preferred_element_type=jnp.float32)
        m1b = z if m1b is None else jnp.maximum(m1b, z)

    feat = jnp.concatenate([m1a, m1b], axis=0) + bias_ref[...]
    feat_ref[...] = feat.astype(feat_ref.dtype)

    ones = jnp.ones((tn, 128), jnp.float32)
    s1_ref[...] = jnp.dot(feat, ones,
                          preferred_element_type=jnp.float32)[None]
    s2_ref[...] = jnp.dot(feat * feat, ones,
                          preferred_element_type=jnp.float32)[None]


def _head_kernel(n_real, n_fake, feat_ref, s1_ref, s2_ref, bias_ref,
                 gamma_ref, beta_ref, w2a_ref, b2a_ref, w2b_ref, b2b_ref,
                 wfc_ref, bfc_ref, out_ref):
    bias = bias_ref[...]
    s1 = jnp.sum(s1_ref[...], axis=0)[:, 0:1] - n_fake * bias
    s2 = jnp.sum(s2_ref[...], axis=0)[:, 0:1] - n_fake * bias * bias
    mean = s1 / n_real
    var = jnp.maximum(s2 / n_real - mean * mean, 0.0)
    inv = jax.lax.rsqrt(var + _BN_EPS)
    scale = gamma_ref[...] * inv
    shift = beta_ref[...] - mean * scale

    feat = feat_ref[...].astype(jnp.float32)
    h = jnp.maximum(feat * scale + shift, 0.0)
    h = jnp.dot(w2a_ref[...], h.astype(jnp.bfloat16),
                preferred_element_type=jnp.float32) + b2a_ref[...]
    h = jnp.maximum(h, 0.0)
    h = jnp.dot(w2b_ref[...], h.astype(jnp.bfloat16),
                preferred_element_type=jnp.float32) + b2b_ref[...]
    res = feat + h
    out = jnp.dot(wfc_ref[...], res.astype(jnp.bfloat16),
                  preferred_element_type=jnp.float32) + bfc_ref[...]
    out_ref[...] = out[0:_NUM_CLASSES].astype(out_ref.dtype)


@functools.partial(jax.jit, static_argnames=("tile_n",))
def _forward(x, w1a, b1a, w1b, b1b, gamma, beta, w2a, b2a, w2b, b2b,
             wfc, bfc, tile_n=512):
    n = x.shape[0]
    tn = min(tile_n, _round_up(n, 128))
    n_pad = _round_up(n, tn)
    grid_n = n_pad // tn

    xt = x.transpose(2, 3, 1, 0).reshape(_PP, _IN_CH, n)
    if n_pad != n:
        xt = jnp.pad(xt, ((0, 0), (0, 0), (0, n_pad - n)))

    w1a_r = w1a.transpose(2, 3, 1, 0).reshape(9, _IN_CH, 64)
    ind = jnp.asarray(_conv1a_indicator())
    w1a_full = jnp.einsum("tcm,tpo->pcom", w1a_r, ind)
    w1a_full = w1a_full.reshape(_PP, _IN_CH, 576)
    w1a_full = jnp.pad(w1a_full, ((0, 0), (0, _CPAD - _IN_CH), (0, 0)))
    w1a_full = w1a_full.reshape(_PP * _CPAD, 576).astype(jnp.bfloat16)
    w1b_e = w1b.reshape(64, _IN_CH).T.astype(jnp.bfloat16)

    bias = jnp.concatenate([b1a, b1b]).reshape(128, 1).astype(jnp.float32)
    gamma_t = gamma.reshape(128, 1).astype(jnp.float32)
    beta_t = beta.reshape(128, 1).astype(jnp.float32)
    w2a_m = w2a.reshape(128, 128).astype(jnp.bfloat16)
    w2b_m = w2b.reshape(128, 128).astype(jnp.bfloat16)
    b2a_t = b2a.reshape(128, 1).astype(jnp.float32)
    b2b_t = b2b.reshape(128, 1).astype(jnp.float32)
    wfc_p = jnp.pad(wfc, ((0, 128 - _NUM_CLASSES), (0, 0))).astype(jnp.bfloat16)
    bfc_p = jnp.pad(bfc, (0, 128 - _NUM_CLASSES)).reshape(128, 1).astype(jnp.float32)

    cparams = pltpu.CompilerParams(
        dimension_semantics=("parallel",),
        vmem_limit_bytes=100 << 20,
    )

    feat, s1, s2 = pl.pallas_call(
        _feat_kernel,
        out_shape=(
            jax.ShapeDtypeStruct((128, n_pad), jnp.bfloat16),
            jax.ShapeDtypeStruct((grid_n, 128, 128), jnp.float32),
            jax.ShapeDtypeStruct((grid_n, 128, 128), jnp.float32),
        ),
        grid=(grid_n,),
        in_specs=[
            pl.BlockSpec((_PP, _IN_CH, tn), lambda i: (0, 0, i)),
            pl.BlockSpec((_PP * _CPAD, 576), lambda i: (0, 0)),
            pl.BlockSpec((_IN_CH, 64), lambda i: (0, 0)),
            pl.BlockSpec((128, 1), lambda i: (0, 0)),
        ],
        out_specs=(
            pl.BlockSpec((128, tn), lambda i: (0, i)),
            pl.BlockSpec((1, 128, 128), lambda i: (i, 0, 0)),
            pl.BlockSpec((1, 128, 128), lambda i: (i, 0, 0)),
        ),
        scratch_shapes=[pltpu.VMEM((_PP * _CPAD, tn), jnp.bfloat16)],
        compiler_params=cparams,
    )(xt, w1a_full, w1b_e, bias)

    kern = functools.partial(_head_kernel, float(n), float(n_pad - n))
    out_t = pl.pallas_call(
        kern,
        out_shape=jax.ShapeDtypeStruct((_NUM_CLASSES, n_pad), jnp.bfloat16),
        grid=(grid_n,),
        in_specs=[
            pl.BlockSpec((128, tn), lambda i: (0, i)),
            pl.BlockSpec((grid_n, 128, 128), lambda i: (0, 0, 0)),
            pl.BlockSpec((grid_n, 128, 128), lambda i: (0, 0, 0)),
            pl.BlockSpec((128, 1), lambda i: (0, 0)),
            pl.BlockSpec((128, 1), lambda i: (0, 0)),
            pl.BlockSpec((128, 1), lambda i: (0, 0)),
            pl.BlockSpec((128, 128), lambda i: (0, 0)),
            pl.BlockSpec((128, 1), lambda i: (0, 0)),
            pl.BlockSpec((128, 128), lambda i: (0, 0)),
            pl.BlockSpec((128, 1), lambda i: (0, 0)),
            pl.BlockSpec((128, 128), lambda i: (0, 0)),
            pl.BlockSpec((128, 1), lambda i: (0, 0)),
        ],
        out_specs=pl.BlockSpec((_NUM_CLASSES, tn), lambda i: (0, i)),
        compiler_params=cparams,
    )(feat, s1, s2, bias, gamma_t, beta_t, w2a_m, b2a_t, w2b_m, b2b_t,
      wfc_p, bfc_p)

    return out_t[:, :n].T.astype(jnp.float32)


def kernel(x, w1a, b1a, w1b, b1b, gamma, beta, w2a, b2a, w2b, b2b, wfc, bfc):
    return _forward(x, w1a, b1a, w1b, b1b, gamma, beta, w2a, b2a, w2b, b2b,
                    wfc, bfc, tile_n=512)


# speedup vs baseline: 5.9869x; 1.0988x over previous
import functools

import numpy as np

import jax
import jax.numpy as jnp
from jax.experimental import pallas as pl
from jax.experimental.pallas import tpu as pltpu

_NUM_CLASSES = 9
_IN_CH = 103
_P = 5
_PP = _P * _P
_BN_EPS = 1e-5


def _round_up(v, m):
    return (v + m - 1) // m * m


def _conv1a_indicator():
    ind = np.zeros((9, _PP, 9), np.float32)
    for oi in range(3):
        for oj in range(3):
            o = oi * 3 + oj
            for ki in range(3):
                for kj in range(3):
                    ind[ki * 3 + kj, (oi + ki) * _P + (oj + kj), o] = 1.0
    return ind


_CPAD = 112


def _feat_kernel(x_ref, w1a_ref, w1b_ref, bias_ref,
                 feat_ref, s1_ref, s2_ref, xs_ref):
    tn = feat_ref.shape[1]

    zpad = jnp.zeros((_CPAD - _IN_CH, tn), jnp.bfloat16)
    xb = []
    for p in range(_PP):
        xp = x_ref[p].astype(jnp.bfloat16)
        xb.append(xp)
        xs_ref[p * _CPAD:p * _CPAD + _IN_CH, :] = xp
        xs_ref[p * _CPAD + _IN_CH:(p + 1) * _CPAD, :] = zpad
    xs = xs_ref[...]

    y1a = jnp.dot(w1a_ref[...], xs,
                  preferred_element_type=jnp.float32)
    m1a = y1a[0:64]
    for o in range(1, 9):
        m1a = jnp.maximum(m1a, y1a[o * 64:(o + 1) * 64])

    w1b = w1b_ref[...]
    m1b = None
    for p in range(_PP):
        z = jnp.dot(w1b, xb[p],
                    preferred_element_type=jnp.float32)
        m1b = z if m1b is None else jnp.maximum(m1b, z)

    feat = jnp.concatenate([m1a, m1b], axis=0) + bias_ref[...]
    feat_ref[...] = feat.astype(feat_ref.dtype)

    ones = jnp.ones((tn, 128), jnp.float32)
    s1_ref[...] = jnp.dot(feat, ones,
                          preferred_element_type=jnp.float32)[None]
    s2_ref[...] = jnp.dot(feat * feat, ones,
                          preferred_element_type=jnp.float32)[None]


def _head_kernel(n_real, n_fake, feat_ref, s1_ref, s2_ref, bias_ref,
                 gamma_ref, beta_ref, w2a_ref, b2a_ref, w2b_ref, b2b_ref,
                 wfc_ref, bfc_ref, out_ref):
    bias = bias_ref[...]
    s1 = jnp.sum(s1_ref[...], axis=0)[:, 0:1] - n_fake * bias
    s2 = jnp.sum(s2_ref[...], axis=0)[:, 0:1] - n_fake * bias * bias
    mean = s1 / n_real
    var = jnp.maximum(s2 / n_real - mean * mean, 0.0)
    inv = jax.lax.rsqrt(var + _BN_EPS)
    scale = gamma_ref[...] * inv
    shift = beta_ref[...] - mean * scale

    feat = feat_ref[...].astype(jnp.float32)
    h = jnp.maximum(feat * scale + shift, 0.0)
    h = jnp.dot(w2a_ref[...], h.astype(jnp.bfloat16),
                preferred_element_type=jnp.float32) + b2a_ref[...]
    h = jnp.maximum(h, 0.0)
    h = jnp.dot(w2b_ref[...], h.astype(jnp.bfloat16),
                preferred_element_type=jnp.float32) + b2b_ref[...]
    res = feat + h
    out = jnp.dot(wfc_ref[...], res.astype(jnp.bfloat16),
                  preferred_element_type=jnp.float32) + bfc_ref[...]
    out_ref[...] = out[0:_NUM_CLASSES].astype(out_ref.dtype)


@functools.partial(jax.jit, static_argnames=("tile_n",))
def _forward(x, w1a, b1a, w1b, b1b, gamma, beta, w2a, b2a, w2b, b2b,
             wfc, bfc, tile_n=1024):
    n = x.shape[0]
    tn = min(tile_n, _round_up(n, 128))
    n_pad = _round_up(n, tn)
    grid_n = n_pad // tn

    xt = x.transpose(2, 3, 1, 0).reshape(_PP, _IN_CH, n)
    if n_pad != n:
        xt = jnp.pad(xt, ((0, 0), (0, 0), (0, n_pad - n)))

    w1a_r = w1a.transpose(2, 3, 1, 0).reshape(9, _IN_CH, 64)
    ind = jnp.asarray(_conv1a_indicator())
    w1a_full = jnp.einsum("tcm,tpo->ompc", w1a_r, ind)
    w1a_full = jnp.pad(w1a_full, ((0, 0), (0, 0), (0, 0), (0, _CPAD - _IN_CH)))
    w1a_full = w1a_full.reshape(576, _PP * _CPAD).astype(jnp.bfloat16)
    w1b_e = w1b.reshape(64, _IN_CH).astype(jnp.bfloat16)

    bias = jnp.concatenate([b1a, b1b]).reshape(128, 1).astype(jnp.float32)
    gamma_t = gamma.reshape(128, 1).astype(jnp.float32)
    beta_t = beta.reshape(128, 1).astype(jnp.float32)
    w2a_m = w2a.reshape(128, 128).astype(jnp.bfloat16)
    w2b_m = w2b.reshape(128, 128).astype(jnp.bfloat16)
    b2a_t = b2a.reshape(128, 1).astype(jnp.float32)
    b2b_t = b2b.reshape(128, 1).astype(jnp.float32)
    wfc_p = jnp.pad(wfc, ((0, 128 - _NUM_CLASSES), (0, 0))).astype(jnp.bfloat16)
    bfc_p = jnp.pad(bfc, (0, 128 - _NUM_CLASSES)).reshape(128, 1).astype(jnp.float32)

    cparams = pltpu.CompilerParams(
        dimension_semantics=("parallel",),
        vmem_limit_bytes=100 << 20,
    )

    feat, s1, s2 = pl.pallas_call(
        _feat_kernel,
        out_shape=(
            jax.ShapeDtypeStruct((128, n_pad), jnp.bfloat16),
            jax.ShapeDtypeStruct((grid_n, 128, 128), jnp.float32),
            jax.ShapeDtypeStruct((grid_n, 128, 128), jnp.float32),
        ),
        grid=(grid_n,),
        in_specs=[
            pl.BlockSpec((_PP, _IN_CH, tn), lambda i: (0, 0, i)),
            pl.BlockSpec((576, _PP * _CPAD), lambda i: (0, 0)),
            pl.BlockSpec((64, _IN_CH), lambda i: (0, 0)),
            pl.BlockSpec((128, 1), lambda i: (0, 0)),
        ],
        out_specs=(
            pl.BlockSpec((128, tn), lambda i: (0, i)),
            pl.BlockSpec((1, 128, 128), lambda i: (i, 0, 0)),
            pl.BlockSpec((1, 128, 128), lambda i: (i, 0, 0)),
        ),
        scratch_shapes=[pltpu.VMEM((_PP * _CPAD, tn), jnp.bfloat16)],
        compiler_params=cparams,
    )(xt, w1a_full, w1b_e, bias)

    kern = functools.partial(_head_kernel, float(n), float(n_pad - n))
    out_t = pl.pallas_call(
        kern,
        out_shape=jax.ShapeDtypeStruct((_NUM_CLASSES, n_pad), jnp.bfloat16),
        grid=(grid_n,),
        in_specs=[
            pl.BlockSpec((128, tn), lambda i: (0, i)),
            pl.BlockSpec((grid_n, 128, 128), lambda i: (0, 0, 0)),
            pl.BlockSpec((grid_n, 128, 128), lambda i: (0, 0, 0)),
            pl.BlockSpec((128, 1), lambda i: (0, 0)),
            pl.BlockSpec((128, 1), lambda i: (0, 0)),
            pl.BlockSpec((128, 1), lambda i: (0, 0)),
            pl.BlockSpec((128, 128), lambda i: (0, 0)),
            pl.BlockSpec((128, 1), lambda i: (0, 0)),
            pl.BlockSpec((128, 128), lambda i: (0, 0)),
            pl.BlockSpec((128, 1), lambda i: (0, 0)),
            pl.BlockSpec((128, 128), lambda i: (0, 0)),
            pl.BlockSpec((128, 1), lambda i: (0, 0)),
        ],
        out_specs=pl.BlockSpec((_NUM_CLASSES, tn), lambda i: (0, i)),
        compiler_params=cparams,
    )(feat, s1, s2, bias, gamma_t, beta_t, w2a_m, b2a_t, w2b_m, b2b_t,
      wfc_p, bfc_p)

    return out_t[:, :n].T.astype(jnp.float32)


def kernel(x, w1a, b1a, w1b, b1b, gamma, beta, w2a, b2a, w2b, b2b, wfc, bfc):
    return _forward(x, w1a, b1a, w1b, b1b, gamma, beta, w2a, b2a, w2b, b2b,
                    wfc, bfc, tile_n=1024)


# speedup vs baseline: 7.9063x; 1.3206x over previous
import functools

import numpy as np

import jax
import jax.numpy as jnp
from jax.experimental import pallas as pl
from jax.experimental.pallas import tpu as pltpu

_NUM_CLASSES = 9
_IN_CH = 103
_P = 5
_PP = _P * _P
_CPAD = 128
_KF = _PP * _CPAD
_BN_EPS = 1e-5


def _round_up(v, m):
    return (v + m - 1) // m * m


def _tap_map():
    taps = {}
    for oi in range(3):
        for oj in range(3):
            for ki in range(3):
                for kj in range(3):
                    taps[(oi * 3 + oj, (oi + ki) * _P + (oj + kj))] = ki * 3 + kj
    return taps


_TAPS = _tap_map()


def _feat_kernel(x_ref, pa_ref, pb_ref, ps_ref,
                 feat_ref, s1_ref, s2_ref, xs_ref, w_ref):
    tn = feat_ref.shape[1]
    j = pl.program_id(1)

    @pl.when(j == 0)
    def _build_w():
        w_ref[...] = jnp.zeros_like(w_ref)
        for (o, p), t in _TAPS.items():
            w_ref[o * 64:(o + 1) * 64, p * _CPAD:(p + 1) * _CPAD] = (
                pa_ref[t * 64:(t + 1) * 64, :])

    zpad = jnp.zeros((_CPAD - _IN_CH, tn), jnp.bfloat16)
    for p in range(_PP):
        xs_ref[p * _CPAD:p * _CPAD + _IN_CH, :] = x_ref[p].astype(jnp.bfloat16)
        xs_ref[p * _CPAD + _IN_CH:(p + 1) * _CPAD, :] = zpad
    xs = xs_ref[...]

    y1a = jnp.dot(w_ref[...], xs,
                  preferred_element_type=jnp.float32)
    m1a = y1a[0:64]
    for o in range(1, 9):
        m1a = jnp.maximum(m1a, y1a[o * 64:(o + 1) * 64])

    w1b = pb_ref[3, 0:64, :]
    m1b = None
    for p in range(_PP):
        z = jnp.dot(w1b, xs_ref[p * _CPAD:(p + 1) * _CPAD, :],
                    preferred_element_type=jnp.float32)
        m1b = z if m1b is None else jnp.maximum(m1b, z)

    feat = jnp.concatenate([m1a, m1b], axis=0) + ps_ref[:, 0:1]
    feat_ref[...] = feat.astype(feat_ref.dtype)

    ones = jnp.ones((tn, 128), jnp.float32)
    s1_ref[...] = jnp.dot(feat, ones,
                          preferred_element_type=jnp.float32)[None]
    s2_ref[...] = jnp.dot(feat * feat, ones,
                          preferred_element_type=jnp.float32)[None]


def _head_kernel(n_real, n_fake, feat_ref, s1_ref, s2_ref, pb_ref, ps_ref,
                 out_ref):
    bias = ps_ref[:, 0:1]
    s1 = jnp.sum(s1_ref[...], axis=0)[:, 0:1] - n_fake * bias
    s2 = jnp.sum(s2_ref[...], axis=0)[:, 0:1] - n_fake * bias * bias
    mean = s1 / n_real
    var = jnp.maximum(s2 / n_real - mean * mean, 0.0)
    inv = jax.lax.rsqrt(var + _BN_EPS)
    scale = ps_ref[:, 1:2] * inv
    shift = ps_ref[:, 2:3] - mean * scale

    feat = feat_ref[...].astype(jnp.float32)
    h = jnp.maximum(feat * scale + shift, 0.0)
    h = jnp.dot(pb_ref[0], h.astype(jnp.bfloat16),
                preferred_element_type=jnp.float32) + ps_ref[:, 3:4]
    h = jnp.maximum(h, 0.0)
    h = jnp.dot(pb_ref[1], h.astype(jnp.bfloat16),
                preferred_element_type=jnp.float32) + ps_ref[:, 4:5]
    res = feat + h
    out = jnp.dot(pb_ref[2], res.astype(jnp.bfloat16),
                  preferred_element_type=jnp.float32) + ps_ref[:, 5:6]
    out_bf = out[0:_NUM_CLASSES].astype(jnp.bfloat16)
    out_ref[...] = out_bf.astype(jnp.float32)


@functools.partial(jax.jit, static_argnames=("tile_n",))
def _forward(x, w1a, b1a, w1b, b1b, gamma, beta, w2a, b2a, w2b, b2b,
             wfc, bfc, tile_n=1024):
    n = x.shape[0]
    tn = min(tile_n, _round_up(n, 128))
    n_pad = _round_up(n, tn)
    grid_n = n_pad // tn
    ncores = 2 if grid_n % 2 == 0 else 1
    g2 = grid_n // ncores

    xt = x.transpose(2, 3, 1, 0).reshape(_PP, _IN_CH, n)
    if n_pad != n:
        xt = jnp.pad(xt, ((0, 0), (0, 0), (0, n_pad - n)))

    pack_a = w1a.transpose(2, 3, 0, 1).reshape(576, _IN_CH)
    pack_a = jnp.pad(pack_a, ((0, 0), (0, _CPAD - _IN_CH))).astype(jnp.bfloat16)

    w2a_m = w2a.reshape(128, 128).astype(jnp.bfloat16)
    w2b_m = w2b.reshape(128, 128).astype(jnp.bfloat16)
    wfc_p = jnp.pad(wfc.astype(jnp.bfloat16), ((0, 128 - _NUM_CLASSES), (0, 0)))
    w1b_p = jnp.pad(w1b.reshape(64, _IN_CH).astype(jnp.bfloat16),
                    ((0, 64), (0, _CPAD - _IN_CH)))
    pack_b = jnp.stack([w2a_m, w2b_m, wfc_p, w1b_p])

    bias = jnp.concatenate([b1a, b1b])
    bfc_p = jnp.pad(bfc, (0, 128 - _NUM_CLASSES))
    zcol = jnp.zeros((128,), jnp.float32)
    pack_s = jnp.stack([bias, gamma, beta, b2a, b2b, bfc_p, zcol, zcol],
                       axis=1).astype(jnp.float32)

    cparams1 = pltpu.CompilerParams(
        dimension_semantics=("parallel", "arbitrary"),
        vmem_limit_bytes=100 << 20,
    )
    cparams2 = pltpu.CompilerParams(
        dimension_semantics=("parallel",),
        vmem_limit_bytes=100 << 20,
    )

    feat, s1, s2 = pl.pallas_call(
        _feat_kernel,
        out_shape=(
            jax.ShapeDtypeStruct((128, n_pad), jnp.bfloat16),
            jax.ShapeDtypeStruct((grid_n, 128, 128), jnp.float32),
            jax.ShapeDtypeStruct((grid_n, 128, 128), jnp.float32),
        ),
        grid=(ncores, g2),
        in_specs=[
            pl.BlockSpec((_PP, _IN_CH, tn), lambda c, j: (0, 0, c * g2 + j)),
            pl.BlockSpec((576, _CPAD), lambda c, j: (0, 0)),
            pl.BlockSpec((4, 128, 128), lambda c, j: (0, 0, 0)),
            pl.BlockSpec((128, 8), lambda c, j: (0, 0)),
        ],
        out_specs=(
            pl.BlockSpec((128, tn), lambda c, j: (0, c * g2 + j)),
            pl.BlockSpec((1, 128, 128), lambda c, j: (c * g2 + j, 0, 0)),
            pl.BlockSpec((1, 128, 128), lambda c, j: (c * g2 + j, 0, 0)),
        ),
        scratch_shapes=[
            pltpu.VMEM((_KF, tn), jnp.bfloat16),
            pltpu.VMEM((576, _KF), jnp.bfloat16),
        ],
        compiler_params=cparams1,
    )(xt, pack_a, pack_b, pack_s)

    kern = functools.partial(_head_kernel, float(n), float(n_pad - n))
    out_t = pl.pallas_call(
        kern,
        out_shape=jax.ShapeDtypeStruct((_NUM_CLASSES, n_pad), jnp.float32),
        grid=(grid_n,),
        in_specs=[
            pl.BlockSpec((128, tn), lambda i: (0, i)),
            pl.BlockSpec((grid_n, 128, 128), lambda i: (0, 0, 0)),
            pl.BlockSpec((grid_n, 128, 128), lambda i: (0, 0, 0)),
            pl.BlockSpec((4, 128, 128), lambda i: (0, 0, 0)),
            pl.BlockSpec((128, 8), lambda i: (0, 0)),
        ],
        out_specs=pl.BlockSpec((_NUM_CLASSES, tn), lambda i: (0, i)),
        compiler_params=cparams2,
    )(feat, s1, s2, pack_b, pack_s)

    return out_t[:, :n].T


def kernel(x, w1a, b1a, w1b, b1b, gamma, beta, w2a, b2a, w2b, b2b, wfc, bfc):
    return _forward(x, w1a, b1a, w1b, b1b, gamma, beta, w2a, b2a, w2b, b2b,
                    wfc, bfc, tile_n=1024)


# speedup vs baseline: 8.5688x; 1.0838x over previous
import functools

import jax
import jax.numpy as jnp
from jax.experimental import pallas as pl
from jax.experimental.pallas import tpu as pltpu

_NUM_CLASSES = 9
_IN_CH = 103
_P = 5
_PP = _P * _P
_CPAD = 128
_KF = _PP * _CPAD
_BN_EPS = 1e-5


def _round_up(v, m):
    return (v + m - 1) // m * m


def _tap_map():
    taps = {}
    for oi in range(3):
        for oj in range(3):
            for ki in range(3):
                for kj in range(3):
                    taps[(oi * 3 + oj, (oi + ki) * _P + (oj + kj))] = ki * 3 + kj
    return taps


_TAPS = _tap_map()


def _feat_kernel(x_ref, pa_ref, w1b_ref, ps_ref,
                 feat_ref, s1_ref, s2_ref, xs_ref, w_ref):
    tn = feat_ref.shape[1]
    j = pl.program_id(1)

    @pl.when(j == 0)
    def _build_w():
        w_ref[...] = jnp.zeros_like(w_ref)
        for (o, p), t in _TAPS.items():
            w_ref[o * 64:(o + 1) * 64, p * _CPAD:(p + 1) * _CPAD] = (
                pa_ref[t * 64:(t + 1) * 64, :])

    zpad = jnp.zeros((_CPAD - _IN_CH, tn), jnp.bfloat16)
    for p in range(_PP):
        xs_ref[p * _CPAD:p * _CPAD + _IN_CH, :] = x_ref[p].astype(jnp.bfloat16)
        xs_ref[p * _CPAD + _IN_CH:(p + 1) * _CPAD, :] = zpad
    xs = xs_ref[...]

    y1a = jnp.dot(w_ref[...], xs,
                  preferred_element_type=jnp.float32)
    m1a = y1a[0:64]
    for o in range(1, 9):
        m1a = jnp.maximum(m1a, y1a[o * 64:(o + 1) * 64])

    w1b = w1b_ref[...].astype(jnp.bfloat16)
    m1b = None
    for p in range(_PP):
        z = jnp.dot(w1b, xs_ref[p * _CPAD:p * _CPAD + _IN_CH, :],
                    preferred_element_type=jnp.float32)
        m1b = z if m1b is None else jnp.maximum(m1b, z)

    feat = jnp.concatenate([m1a, m1b], axis=0) + ps_ref[:, 0:1]
    feat_ref[...] = feat.astype(feat_ref.dtype)

    ones = jnp.ones((tn, 128), jnp.float32)
    s1_ref[...] = jnp.dot(feat, ones,
                          preferred_element_type=jnp.float32)[None]
    s2_ref[...] = jnp.dot(feat * feat, ones,
                          preferred_element_type=jnp.float32)[None]


def _head_kernel(n_real, n_fake, feat_ref, s1_ref, s2_ref, ps_ref,
                 w2a_ref, w2b_ref, wfc_ref, out_ref):
    bias = ps_ref[:, 0:1]
    s1 = jnp.sum(s1_ref[...], axis=0)[:, 0:1] - n_fake * bias
    s2 = jnp.sum(s2_ref[...], axis=0)[:, 0:1] - n_fake * bias * bias
    mean = s1 / n_real
    var = jnp.maximum(s2 / n_real - mean * mean, 0.0)
    inv = jax.lax.rsqrt(var + _BN_EPS)
    scale = ps_ref[:, 1:2] * inv
    shift = ps_ref[:, 2:3] - mean * scale

    feat = feat_ref[...].astype(jnp.float32)
    h = jnp.maximum(feat * scale + shift, 0.0)
    h = jnp.dot(w2a_ref[...].astype(jnp.bfloat16), h.astype(jnp.bfloat16),
                preferred_element_type=jnp.float32) + ps_ref[:, 3:4]
    h = jnp.maximum(h, 0.0)
    h = jnp.dot(w2b_ref[...].astype(jnp.bfloat16), h.astype(jnp.bfloat16),
                preferred_element_type=jnp.float32) + ps_ref[:, 4:5]
    res = feat + h
    out = jnp.dot(wfc_ref[...].astype(jnp.bfloat16), res.astype(jnp.bfloat16),
                  preferred_element_type=jnp.float32) + ps_ref[0:_NUM_CLASSES, 5:6]
    out_bf = out.astype(jnp.bfloat16)
    out_ref[...] = out_bf.astype(jnp.float32)


@functools.partial(jax.jit, static_argnames=("tile_n",))
def _forward(x, w1a, b1a, w1b, b1b, gamma, beta, w2a, b2a, w2b, b2b,
             wfc, bfc, tile_n=1024):
    n = x.shape[0]
    tn = min(tile_n, _round_up(n, 128))
    n_pad = _round_up(n, tn)
    grid_n = n_pad // tn
    ncores = 2 if grid_n % 2 == 0 else 1
    g2 = grid_n // ncores

    xt = x.transpose(2, 3, 1, 0).reshape(_PP, _IN_CH, n)
    if n_pad != n:
        xt = jnp.pad(xt, ((0, 0), (0, 0), (0, n_pad - n)))

    pack_a = w1a.transpose(2, 3, 0, 1).reshape(576, _IN_CH)
    pack_a = jnp.pad(pack_a, ((0, 0), (0, _CPAD - _IN_CH))).astype(jnp.bfloat16)

    w1b_v = w1b.reshape(64, _IN_CH)
    w2a_v = w2a.reshape(128, 128)
    w2b_v = w2b.reshape(128, 128)
    bias = jnp.concatenate([b1a, b1b])
    bfc_p = jnp.pad(bfc, (0, 128 - _NUM_CLASSES))
    zcol = jnp.zeros((128,), jnp.float32)
    pack_s = jnp.stack([bias, gamma, beta, b2a, b2b, bfc_p, zcol, zcol],
                       axis=1)

    cparams1 = pltpu.CompilerParams(
        dimension_semantics=("parallel", "arbitrary"),
        vmem_limit_bytes=100 << 20,
    )
    cparams2 = pltpu.CompilerParams(
        dimension_semantics=("parallel",),
        vmem_limit_bytes=100 << 20,
    )

    feat, s1, s2 = pl.pallas_call(
        _feat_kernel,
        out_shape=(
            jax.ShapeDtypeStruct((128, n_pad), jnp.bfloat16),
            jax.ShapeDtypeStruct((grid_n, 128, 128), jnp.float32),
            jax.ShapeDtypeStruct((grid_n, 128, 128), jnp.float32),
        ),
        grid=(ncores, g2),
        in_specs=[
            pl.BlockSpec((_PP, _IN_CH, tn), lambda c, j: (0, 0, c * g2 + j)),
            pl.BlockSpec((576, _CPAD), lambda c, j: (0, 0)),
            pl.BlockSpec((64, _IN_CH), lambda c, j: (0, 0)),
            pl.BlockSpec((128, 8), lambda c, j: (0, 0)),
        ],
        out_specs=(
            pl.BlockSpec((128, tn), lambda c, j: (0, c * g2 + j)),
            pl.BlockSpec((1, 128, 128), lambda c, j: (c * g2 + j, 0, 0)),
            pl.BlockSpec((1, 128, 128), lambda c, j: (c * g2 + j, 0, 0)),
        ),
        scratch_shapes=[
            pltpu.VMEM((_KF, tn), jnp.bfloat16),
            pltpu.VMEM((576, _KF), jnp.bfloat16),
        ],
        compiler_params=cparams1,
    )(xt, pack_a, w1b_v, pack_s)

    kern = functools.partial(_head_kernel, float(n), float(n_pad - n))
    out_t = pl.pallas_call(
        kern,
        out_shape=jax.ShapeDtypeStruct((_NUM_CLASSES, n_pad), jnp.float32),
        grid=(grid_n,),
        in_specs=[
            pl.BlockSpec((128, tn), lambda i: (0, i)),
            pl.BlockSpec((grid_n, 128, 128), lambda i: (0, 0, 0)),
            pl.BlockSpec((grid_n, 128, 128), lambda i: (0, 0, 0)),
            pl.BlockSpec((128, 8), lambda i: (0, 0)),
            pl.BlockSpec((128, 128), lambda i: (0, 0)),
            pl.BlockSpec((128, 128), lambda i: (0, 0)),
            pl.BlockSpec((_NUM_CLASSES, 128), lambda i: (0, 0)),
        ],
        out_specs=pl.BlockSpec((_NUM_CLASSES, tn), lambda i: (0, i)),
        compiler_params=cparams2,
    )(feat, s1, s2, pack_s, w2a_v, w2b_v, wfc)

    return out_t[:, :n].T


def kernel(x, w1a, b1a, w1b, b1b, gamma, beta, w2a, b2a, w2b, b2b, wfc, bfc):
    return _forward(x, w1a, b1a, w1b, b1b, gamma, beta, w2a, b2a, w2b, b2b,
                    wfc, bfc, tile_n=1024)


# speedup vs baseline: 9.0329x; 1.0542x over previous
import functools

import jax
import jax.numpy as jnp
from jax.experimental import pallas as pl
from jax.experimental.pallas import tpu as pltpu

_NUM_CLASSES = 9
_IN_CH = 103
_P = 5
_PP = _P * _P
_CPAD = 128
_KF = _PP * _CPAD
_BN_EPS = 1e-5


def _round_up(v, m):
    return (v + m - 1) // m * m


def _tap_map():
    taps = {}
    for oi in range(3):
        for oj in range(3):
            for ki in range(3):
                for kj in range(3):
                    taps[(oi * 3 + oj, (oi + ki) * _P + (oj + kj))] = ki * 3 + kj
    return taps


_TAPS = _tap_map()


def _feat_kernel(x_ref, pa_ref, w1b_ref, ps_ref,
                 feat_ref, s1_ref, s2_ref, xs_ref, w_ref):
    tn = feat_ref.shape[1]
    j = pl.program_id(1)

    @pl.when(j == 0)
    def _build_w():
        w_ref[...] = jnp.zeros_like(w_ref)
        for t in range(9):
            blk = pa_ref[t * 64:(t + 1) * 64, :].astype(jnp.bfloat16)
            for (o, p), tt in _TAPS.items():
                if tt == t:
                    w_ref[o * 64:(o + 1) * 64,
                          p * _CPAD:p * _CPAD + _IN_CH] = blk

    zpad = jnp.zeros((_CPAD - _IN_CH, tn), jnp.bfloat16)
    for p in range(_PP):
        xs_ref[p * _CPAD:p * _CPAD + _IN_CH, :] = x_ref[p].astype(jnp.bfloat16)
        xs_ref[p * _CPAD + _IN_CH:(p + 1) * _CPAD, :] = zpad
    xs = xs_ref[...]

    y1a = jnp.dot(w_ref[...], xs,
                  preferred_element_type=jnp.float32)
    m1a = y1a[0:64]
    for o in range(1, 9):
        m1a = jnp.maximum(m1a, y1a[o * 64:(o + 1) * 64])

    w1b = w1b_ref[...].astype(jnp.bfloat16)
    m1b = None
    for p in range(_PP):
        z = jnp.dot(w1b, xs_ref[p * _CPAD:p * _CPAD + _IN_CH, :],
                    preferred_element_type=jnp.float32)
        m1b = z if m1b is None else jnp.maximum(m1b, z)

    feat = jnp.concatenate([m1a, m1b], axis=0) + ps_ref[:, 0:1]
    feat_ref[...] = feat.astype(feat_ref.dtype)

    ones = jnp.ones((tn, 128), jnp.float32)
    s1_ref[...] = jnp.dot(feat, ones,
                          preferred_element_type=jnp.float32)[None]
    s2_ref[...] = jnp.dot(feat * feat, ones,
                          preferred_element_type=jnp.float32)[None]


def _head_kernel(n_real, n_fake, feat_ref, s1_ref, s2_ref, ps_ref,
                 w2a_ref, w2b_ref, wfc_ref, out_ref):
    bias = ps_ref[:, 0:1]
    s1 = jnp.sum(s1_ref[...], axis=0)[:, 0:1] - n_fake * bias
    s2 = jnp.sum(s2_ref[...], axis=0)[:, 0:1] - n_fake * bias * bias
    mean = s1 / n_real
    var = jnp.maximum(s2 / n_real - mean * mean, 0.0)
    inv = jax.lax.rsqrt(var + _BN_EPS)
    scale = ps_ref[:, 1:2] * inv
    shift = ps_ref[:, 2:3] - mean * scale

    feat = feat_ref[...].astype(jnp.float32)
    h = jnp.maximum(feat * scale + shift, 0.0)
    h = jnp.dot(w2a_ref[...].astype(jnp.bfloat16), h.astype(jnp.bfloat16),
                preferred_element_type=jnp.float32) + ps_ref[:, 3:4]
    h = jnp.maximum(h, 0.0)
    h = jnp.dot(w2b_ref[...].astype(jnp.bfloat16), h.astype(jnp.bfloat16),
                preferred_element_type=jnp.float32) + ps_ref[:, 4:5]
    res = feat + h
    out = jnp.dot(wfc_ref[...].astype(jnp.bfloat16), res.astype(jnp.bfloat16),
                  preferred_element_type=jnp.float32) + ps_ref[0:_NUM_CLASSES, 5:6]
    out_bf = out.astype(jnp.bfloat16)
    out_ref[...] = out_bf.astype(jnp.float32)


@functools.partial(jax.jit, static_argnames=("tile_n",))
def _forward(x, w1a, b1a, w1b, b1b, gamma, beta, w2a, b2a, w2b, b2b,
             wfc, bfc, tile_n=1024):
    n = x.shape[0]
    tn = min(tile_n, _round_up(n, 128))
    n_pad = _round_up(n, tn)
    grid_n = n_pad // tn
    ncores = 2 if grid_n % 2 == 0 else 1
    g2 = grid_n // ncores

    xt = x.transpose(2, 3, 1, 0).reshape(_PP, _IN_CH, n)
    if n_pad != n:
        xt = jnp.pad(xt, ((0, 0), (0, 0), (0, n_pad - n)))

    pack_a = w1a.transpose(2, 3, 0, 1).reshape(576, _IN_CH)

    w1b_v = w1b.reshape(64, _IN_CH)
    w2a_v = w2a.reshape(128, 128)
    w2b_v = w2b.reshape(128, 128)
    bias = jnp.concatenate([b1a, b1b])
    bfc_p = jnp.pad(bfc, (0, 128 - _NUM_CLASSES))
    zcol = jnp.zeros((128,), jnp.float32)
    pack_s = jnp.stack([bias, gamma, beta, b2a, b2b, bfc_p, zcol, zcol],
                       axis=1)

    cparams1 = pltpu.CompilerParams(
        dimension_semantics=("parallel", "arbitrary"),
        vmem_limit_bytes=100 << 20,
    )
    cparams2 = pltpu.CompilerParams(
        dimension_semantics=("parallel",),
        vmem_limit_bytes=100 << 20,
    )

    feat, s1, s2 = pl.pallas_call(
        _feat_kernel,
        out_shape=(
            jax.ShapeDtypeStruct((128, n_pad), jnp.bfloat16),
            jax.ShapeDtypeStruct((grid_n, 128, 128), jnp.float32),
            jax.ShapeDtypeStruct((grid_n, 128, 128), jnp.float32),
        ),
        grid=(ncores, g2),
        in_specs=[
            pl.BlockSpec((_PP, _IN_CH, tn), lambda c, j: (0, 0, c * g2 + j)),
            pl.BlockSpec((576, _IN_CH), lambda c, j: (0, 0)),
            pl.BlockSpec((64, _IN_CH), lambda c, j: (0, 0)),
            pl.BlockSpec((128, 8), lambda c, j: (0, 0)),
        ],
        out_specs=(
            pl.BlockSpec((128, tn), lambda c, j: (0, c * g2 + j)),
            pl.BlockSpec((1, 128, 128), lambda c, j: (c * g2 + j, 0, 0)),
            pl.BlockSpec((1, 128, 128), lambda c, j: (c * g2 + j, 0, 0)),
        ),
        scratch_shapes=[
            pltpu.VMEM((_KF, tn), jnp.bfloat16),
            pltpu.VMEM((576, _KF), jnp.bfloat16),
        ],
        compiler_params=cparams1,
    )(xt, pack_a, w1b_v, pack_s)

    kern = functools.partial(_head_kernel, float(n), float(n_pad - n))
    out_t = pl.pallas_call(
        kern,
        out_shape=jax.ShapeDtypeStruct((_NUM_CLASSES, n_pad), jnp.float32),
        grid=(grid_n,),
        in_specs=[
            pl.BlockSpec((128, tn), lambda i: (0, i)),
            pl.BlockSpec((grid_n, 128, 128), lambda i: (0, 0, 0)),
            pl.BlockSpec((grid_n, 128, 128), lambda i: (0, 0, 0)),
            pl.BlockSpec((128, 8), lambda i: (0, 0)),
            pl.BlockSpec((128, 128), lambda i: (0, 0)),
            pl.BlockSpec((128, 128), lambda i: (0, 0)),
            pl.BlockSpec((_NUM_CLASSES, 128), lambda i: (0, 0)),
        ],
        out_specs=pl.BlockSpec((_NUM_CLASSES, tn), lambda i: (0, i)),
        compiler_params=cparams2,
    )(feat, s1, s2, pack_s, w2a_v, w2b_v, wfc)

    return out_t[:, :n].T


def kernel(x, w1a, b1a, w1b, b1b, gamma, beta, w2a, b2a, w2b, b2b, wfc, bfc):
    return _forward(x, w1a, b1a, w1b, b1b, gamma, beta, w2a, b2a, w2b, b2b,
                    wfc, bfc, tile_n=1024)


# speedup vs baseline: 9.3419x; 1.0342x over previous
import functools

import jax
import jax.numpy as jnp
from jax.experimental import pallas as pl
from jax.experimental.pallas import tpu as pltpu

_NUM_CLASSES = 9
_IN_CH = 103
_P = 5
_PP = _P * _P
_CPAD = 128
_KF = _PP * _CPAD
_BN_EPS = 1e-5


def _round_up(v, m):
    return (v + m - 1) // m * m


def _tap_map():
    taps = {}
    for oi in range(3):
        for oj in range(3):
            for ki in range(3):
                for kj in range(3):
                    taps[(oi * 3 + oj, (oi + ki) * _P + (oj + kj))] = ki * 3 + kj
    return taps


_TAPS = _tap_map()


def _feat_kernel(x_ref, pa_ref, w1b_ref, ps_ref,
                 feat_ref, s1_ref, s2_ref, xs_ref, w_ref):
    tn = feat_ref.shape[1]
    j = pl.program_id(1)

    @pl.when(j == 0)
    def _build_w():
        w_ref[...] = jnp.zeros_like(w_ref)
        for t in range(9):
            blk = pa_ref[t * 64:(t + 1) * 64, :].astype(jnp.bfloat16)
            for (o, p), tt in _TAPS.items():
                if tt == t:
                    w_ref[o * 64:(o + 1) * 64,
                          p * _CPAD:p * _CPAD + _IN_CH] = blk
        zpad = jnp.zeros((_CPAD - _IN_CH, tn), jnp.bfloat16)
        for p in range(_PP):
            xs_ref[p * _CPAD + _IN_CH:(p + 1) * _CPAD, :] = zpad

    for p in range(_PP):
        xs_ref[p * _CPAD:p * _CPAD + _IN_CH, :] = x_ref[p].astype(jnp.bfloat16)
    xs = xs_ref[...]

    y1a = jnp.dot(w_ref[...], xs,
                  preferred_element_type=jnp.float32)
    m1a = y1a[0:64]
    for o in range(1, 9):
        m1a = jnp.maximum(m1a, y1a[o * 64:(o + 1) * 64])

    w1b = w1b_ref[...].astype(jnp.bfloat16)
    m1b = None
    for p in range(_PP):
        z = jnp.dot(w1b, xs_ref[p * _CPAD:p * _CPAD + _IN_CH, :],
                    preferred_element_type=jnp.float32)
        m1b = z if m1b is None else jnp.maximum(m1b, z)

    feat = jnp.concatenate([m1a, m1b], axis=0) + ps_ref[:, 0:1]
    feat_ref[...] = feat.astype(feat_ref.dtype)

    ones = jnp.ones((tn, 128), jnp.float32)
    s1_ref[...] = jnp.dot(feat, ones,
                          preferred_element_type=jnp.float32)[None]
    s2_ref[...] = jnp.dot(feat * feat, ones,
                          preferred_element_type=jnp.float32)[None]


def _head_kernel(n_real, n_fake, feat_ref, s1_ref, s2_ref, ps_ref,
                 w2a_ref, w2b_ref, wfc_ref, out_ref):
    bias = ps_ref[:, 0:1]
    s1 = jnp.sum(s1_ref[...], axis=0)[:, 0:1] - n_fake * bias
    s2 = jnp.sum(s2_ref[...], axis=0)[:, 0:1] - n_fake * bias * bias
    mean = s1 / n_real
    var = jnp.maximum(s2 / n_real - mean * mean, 0.0)
    inv = jax.lax.rsqrt(var + _BN_EPS)
    scale = ps_ref[:, 1:2] * inv
    shift = ps_ref[:, 2:3] - mean * scale

    feat = feat_ref[...].astype(jnp.float32)
    h = jnp.maximum(feat * scale + shift, 0.0)
    h = jnp.dot(w2a_ref[...].astype(jnp.bfloat16), h.astype(jnp.bfloat16),
                preferred_element_type=jnp.float32) + ps_ref[:, 3:4]
    h = jnp.maximum(h, 0.0)
    h = jnp.dot(w2b_ref[...].astype(jnp.bfloat16), h.astype(jnp.bfloat16),
                preferred_element_type=jnp.float32) + ps_ref[:, 4:5]
    res = feat + h
    out = jnp.dot(wfc_ref[...].astype(jnp.bfloat16), res.astype(jnp.bfloat16),
                  preferred_element_type=jnp.float32) + ps_ref[0:_NUM_CLASSES, 5:6]
    out_bf = out.astype(jnp.bfloat16)
    out_ref[...] = out_bf.astype(jnp.float32)


@functools.partial(jax.jit, static_argnames=("tile_n",))
def _forward(x, w1a, b1a, w1b, b1b, gamma, beta, w2a, b2a, w2b, b2b,
             wfc, bfc, tile_n=1024):
    n = x.shape[0]
    tn = min(tile_n, _round_up(n, 128))
    n_pad = _round_up(n, tn)
    grid_n = n_pad // tn
    ncores = 2 if grid_n % 2 == 0 else 1
    g2 = grid_n // ncores

    xt = x.transpose(2, 3, 1, 0).reshape(_PP, _IN_CH, n)
    if n_pad != n:
        xt = jnp.pad(xt, ((0, 0), (0, 0), (0, n_pad - n)))

    pack_a = w1a.transpose(2, 3, 0, 1).reshape(576, _IN_CH)

    w1b_v = w1b.reshape(64, _IN_CH)
    w2a_v = w2a.reshape(128, 128)
    w2b_v = w2b.reshape(128, 128)
    bias = jnp.concatenate([b1a, b1b])
    bfc_p = jnp.pad(bfc, (0, 128 - _NUM_CLASSES))
    zcol = jnp.zeros((128,), jnp.float32)
    pack_s = jnp.stack([bias, gamma, beta, b2a, b2b, bfc_p, zcol, zcol],
                       axis=1)

    cparams1 = pltpu.CompilerParams(
        dimension_semantics=("parallel", "arbitrary"),
        vmem_limit_bytes=100 << 20,
    )
    cparams2 = pltpu.CompilerParams(
        dimension_semantics=("parallel",),
        vmem_limit_bytes=100 << 20,
    )

    feat, s1, s2 = pl.pallas_call(
        _feat_kernel,
        out_shape=(
            jax.ShapeDtypeStruct((128, n_pad), jnp.bfloat16),
            jax.ShapeDtypeStruct((grid_n, 128, 128), jnp.float32),
            jax.ShapeDtypeStruct((grid_n, 128, 128), jnp.float32),
        ),
        grid=(ncores, g2),
        in_specs=[
            pl.BlockSpec((_PP, _IN_CH, tn), lambda c, j: (0, 0, c * g2 + j)),
            pl.BlockSpec((576, _IN_CH), lambda c, j: (0, 0)),
            pl.BlockSpec((64, _IN_CH), lambda c, j: (0, 0)),
            pl.BlockSpec((128, 8), lambda c, j: (0, 0)),
        ],
        out_specs=(
            pl.BlockSpec((128, tn), lambda c, j: (0, c * g2 + j)),
            pl.BlockSpec((1, 128, 128), lambda c, j: (c * g2 + j, 0, 0)),
            pl.BlockSpec((1, 128, 128), lambda c, j: (c * g2 + j, 0, 0)),
        ),
        scratch_shapes=[
            pltpu.VMEM((_KF, tn), jnp.bfloat16),
            pltpu.VMEM((576, _KF), jnp.bfloat16),
        ],
        compiler_params=cparams1,
    )(xt, pack_a, w1b_v, pack_s)

    tn2 = n_pad // ncores if ncores == 2 else tn
    grid_h = n_pad // tn2
    kern = functools.partial(_head_kernel, float(n), float(n_pad - n))
    out_t = pl.pallas_call(
        kern,
        out_shape=jax.ShapeDtypeStruct((_NUM_CLASSES, n_pad), jnp.float32),
        grid=(grid_h,),
        in_specs=[
            pl.BlockSpec((128, tn2), lambda i: (0, i)),
            pl.BlockSpec((grid_n, 128, 128), lambda i: (0, 0, 0)),
            pl.BlockSpec((grid_n, 128, 128), lambda i: (0, 0, 0)),
            pl.BlockSpec((128, 8), lambda i: (0, 0)),
            pl.BlockSpec((128, 128), lambda i: (0, 0)),
            pl.BlockSpec((128, 128), lambda i: (0, 0)),
            pl.BlockSpec((_NUM_CLASSES, 128), lambda i: (0, 0)),
        ],
        out_specs=pl.BlockSpec((_NUM_CLASSES, tn2), lambda i: (0, i)),
        compiler_params=cparams2,
    )(feat, s1, s2, pack_s, w2a_v, w2b_v, wfc)

    return out_t[:, :n].T


def kernel(x, w1a, b1a, w1b, b1b, gamma, beta, w2a, b2a, w2b, b2b, wfc, bfc):
    return _forward(x, w1a, b1a, w1b, b1b, gamma, beta, w2a, b2a, w2b, b2b,
                    wfc, bfc, tile_n=1024)


# speedup vs baseline: 9.6148x; 1.0292x over previous
import functools

import jax
import jax.numpy as jnp
from jax.experimental import pallas as pl
from jax.experimental.pallas import tpu as pltpu

_NUM_CLASSES = 9
_IN_CH = 103
_P = 5
_PP = _P * _P
_CPAD = 128
_KF = _PP * _CPAD
_BN_EPS = 1e-5


def _round_up(v, m):
    return (v + m - 1) // m * m


def _tap_map():
    taps = {}
    for oi in range(3):
        for oj in range(3):
            for ki in range(3):
                for kj in range(3):
                    taps[(oi * 3 + oj, (oi + ki) * _P + (oj + kj))] = ki * 3 + kj
    return taps


_TAPS = _tap_map()


def _feat_kernel(x_ref, pa_ref, w1b_ref, ps_ref,
                 feat_ref, s1_ref, s2_ref, xs_ref, w_ref):
    tn = feat_ref.shape[1]
    j = pl.program_id(1)

    @pl.when(j == 0)
    def _build_w():
        w_ref[...] = jnp.zeros_like(w_ref)
        for t in range(9):
            blk = pa_ref[t * 64:(t + 1) * 64, :].astype(jnp.bfloat16)
            for (o, p), tt in _TAPS.items():
                if tt == t:
                    w_ref[o * 64:(o + 1) * 64,
                          p * _CPAD:p * _CPAD + _IN_CH] = blk
        zpad = jnp.zeros((_CPAD - _IN_CH, tn), jnp.bfloat16)
        for p in range(_PP):
            xs_ref[p * _CPAD + _IN_CH:(p + 1) * _CPAD, :] = zpad

    for p in range(_PP):
        xs_ref[p * _CPAD:p * _CPAD + _IN_CH, :] = x_ref[p].astype(jnp.bfloat16)
    xs = xs_ref[...]

    y1a = jnp.dot(w_ref[...], xs,
                  preferred_element_type=jnp.float32)
    m1a = y1a[0:64]
    for o in range(1, 9):
        m1a = jnp.maximum(m1a, y1a[o * 64:(o + 1) * 64])

    w1b = w1b_ref[...].astype(jnp.bfloat16)
    m1b = None
    for p in range(_PP):
        z = jnp.dot(w1b, xs_ref[p * _CPAD:p * _CPAD + _IN_CH, :],
                    preferred_element_type=jnp.float32)
        m1b = z if m1b is None else jnp.maximum(m1b, z)

    feat = jnp.concatenate([m1a, m1b], axis=0) + ps_ref[:, 0:1]
    feat_ref[...] = feat.astype(feat_ref.dtype)

    ones = jnp.ones((tn, 128), jnp.float32)
    s1_ref[...] = jnp.dot(feat, ones,
                          preferred_element_type=jnp.float32)[None]
    s2_ref[...] = jnp.dot(feat * feat, ones,
                          preferred_element_type=jnp.float32)[None]


def _head_kernel(n_real, n_fake, feat_ref, s1_ref, s2_ref, ps_ref,
                 w2a_ref, w2b_ref, wfc_ref, out_ref):
    bias = ps_ref[:, 0:1]
    s1 = jnp.sum(s1_ref[...], axis=0)[:, 0:1] - n_fake * bias
    s2 = jnp.sum(s2_ref[...], axis=0)[:, 0:1] - n_fake * bias * bias
    mean = s1 / n_real
    var = jnp.maximum(s2 / n_real - mean * mean, 0.0)
    inv = jax.lax.rsqrt(var + _BN_EPS)
    scale = ps_ref[:, 1:2] * inv
    shift = ps_ref[:, 2:3] - mean * scale

    feat = feat_ref[...].astype(jnp.float32)
    h = jnp.maximum(feat * scale + shift, 0.0)
    h = jnp.dot(w2a_ref[...].astype(jnp.bfloat16), h.astype(jnp.bfloat16),
                preferred_element_type=jnp.float32) + ps_ref[:, 3:4]
    h = jnp.maximum(h, 0.0)
    h = jnp.dot(w2b_ref[...].astype(jnp.bfloat16), h.astype(jnp.bfloat16),
                preferred_element_type=jnp.float32) + ps_ref[:, 4:5]
    res = feat + h
    out = jnp.dot(wfc_ref[...].astype(jnp.bfloat16), res.astype(jnp.bfloat16),
                  preferred_element_type=jnp.float32) + ps_ref[0:_NUM_CLASSES, 5:6]
    out_bf = out.astype(jnp.bfloat16)
    out_ref[...] = out_bf.astype(jnp.float32)


@functools.partial(jax.jit, static_argnames=("tile_n",))
def _forward(x, w1a, b1a, w1b, b1b, gamma, beta, w2a, b2a, w2b, b2b,
             wfc, bfc, tile_n=512):
    n = x.shape[0]
    tn = min(tile_n, _round_up(n, 128))
    n_pad = _round_up(n, tn)
    grid_n = n_pad // tn
    ncores = 2 if grid_n % 2 == 0 else 1
    g2 = grid_n // ncores

    xt = x.transpose(2, 3, 1, 0).reshape(_PP, _IN_CH, n)
    if n_pad != n:
        xt = jnp.pad(xt, ((0, 0), (0, 0), (0, n_pad - n)))

    pack_a = w1a.transpose(2, 3, 0, 1).reshape(576, _IN_CH)

    w1b_v = w1b.reshape(64, _IN_CH)
    w2a_v = w2a.reshape(128, 128)
    w2b_v = w2b.reshape(128, 128)
    bias = jnp.concatenate([b1a, b1b])
    bfc_p = jnp.pad(bfc, (0, 128 - _NUM_CLASSES))
    zcol = jnp.zeros((128,), jnp.float32)
    pack_s = jnp.stack([bias, gamma, beta, b2a, b2b, bfc_p, zcol, zcol],
                       axis=1)

    cparams1 = pltpu.CompilerParams(
        dimension_semantics=("parallel", "arbitrary"),
        vmem_limit_bytes=100 << 20,
    )
    cparams2 = pltpu.CompilerParams(
        dimension_semantics=("parallel",),
        vmem_limit_bytes=100 << 20,
    )

    feat, s1, s2 = pl.pallas_call(
        _feat_kernel,
        out_shape=(
            jax.ShapeDtypeStruct((128, n_pad), jnp.bfloat16),
            jax.ShapeDtypeStruct((grid_n, 128, 128), jnp.float32),
            jax.ShapeDtypeStruct((grid_n, 128, 128), jnp.float32),
        ),
        grid=(ncores, g2),
        in_specs=[
            pl.BlockSpec((_PP, _IN_CH, tn), lambda c, j: (0, 0, c * g2 + j)),
            pl.BlockSpec((576, _IN_CH), lambda c, j: (0, 0)),
            pl.BlockSpec((64, _IN_CH), lambda c, j: (0, 0)),
            pl.BlockSpec((128, 8), lambda c, j: (0, 0)),
        ],
        out_specs=(
            pl.BlockSpec((128, tn), lambda c, j: (0, c * g2 + j)),
            pl.BlockSpec((1, 128, 128), lambda c, j: (c * g2 + j, 0, 0)),
            pl.BlockSpec((1, 128, 128), lambda c, j: (c * g2 + j, 0, 0)),
        ),
        scratch_shapes=[
            pltpu.VMEM((_KF, tn), jnp.bfloat16),
            pltpu.VMEM((576, _KF), jnp.bfloat16),
        ],
        compiler_params=cparams1,
    )(xt, pack_a, w1b_v, pack_s)

    tn2 = n_pad // ncores if ncores == 2 else tn
    grid_h = n_pad // tn2
    kern = functools.partial(_head_kernel, float(n), float(n_pad - n))
    out_t = pl.pallas_call(
        kern,
        out_shape=jax.ShapeDtypeStruct((_NUM_CLASSES, n_pad), jnp.float32),
        grid=(grid_h,),
        in_specs=[
            pl.BlockSpec((128, tn2), lambda i: (0, i)),
            pl.BlockSpec((grid_n, 128, 128), lambda i: (0, 0, 0)),
            pl.BlockSpec((grid_n, 128, 128), lambda i: (0, 0, 0)),
            pl.BlockSpec((128, 8), lambda i: (0, 0)),
            pl.BlockSpec((128, 128), lambda i: (0, 0)),
            pl.BlockSpec((128, 128), lambda i: (0, 0)),
            pl.BlockSpec((_NUM_CLASSES, 128), lambda i: (0, 0)),
        ],
        out_specs=pl.BlockSpec((_NUM_CLASSES, tn2), lambda i: (0, i)),
        compiler_params=cparams2,
    )(feat, s1, s2, pack_s, w2a_v, w2b_v, wfc)

    return out_t[:, :n].T


def kernel(x, w1a, b1a, w1b, b1b, gamma, beta, w2a, b2a, w2b, b2b, wfc, bfc):
    return _forward(x, w1a, b1a, w1b, b1b, gamma, beta, w2a, b2a, w2b, b2b,
                    wfc, bfc, tile_n=512)
